# baseline (speedup 1.0000x reference)
import functools

import numpy as np
import jax
import jax.numpy as jnp
from jax import lax
from jax.experimental import pallas as pl
from jax.experimental.pallas import tpu as pltpu
from jax.experimental.pallas import tpu_sc as plsc

D_MODEL = 64
SEQ_LEN = 200
NUM_CORES = 2
NUM_SUBCORES = 16
NUM_WORKERS = NUM_CORES * NUM_SUBCORES
LANES = 16
CHUNK_SEQS = 8


def _positional_encoding(length, dim):
    half = dim // 2
    posn = np.arange(length).reshape(length, 1).astype(np.float32)
    dims = np.arange(half).reshape(1, half).astype(np.float32) / half
    enc = posn / (10000.0 ** dims)
    enc = np.concatenate([np.sin(enc), np.cos(enc)], axis=-1)
    return jnp.asarray(enc, dtype=jnp.float32)


_PE = _positional_encoding(SEQ_LEN, D_MODEL)


@functools.lru_cache(maxsize=None)
def _build(batch):
    seqs_per_worker = batch // NUM_WORKERS
    n_chunks = seqs_per_worker // CHUNK_SEQS
    rows_per_chunk = CHUNK_SEQS * SEQ_LEN
    mesh = plsc.VectorSubcoreMesh(core_axis_name="c", subcore_axis_name="s")

    @functools.partial(
        pl.kernel,
        out_type=jax.ShapeDtypeStruct((batch * SEQ_LEN, D_MODEL), jnp.float32),
        mesh=mesh,
        scratch_types=[
            pltpu.VMEM((rows_per_chunk,), jnp.int32),
            pltpu.VMEM((rows_per_chunk, D_MODEL), jnp.float32),
            pltpu.VMEM((SEQ_LEN, D_MODEL), jnp.float32),
            pltpu.SemaphoreType.DMA,
        ],
        compiler_params=pltpu.CompilerParams(use_tc_tiling_on_sc=False),
    )
    def body(ids_hbm, table_hbm, pe_hbm, out_hbm, idx_v, rows_v, pe_v, sem):
        wid = lax.axis_index("s") * NUM_CORES + lax.axis_index("c")
        pltpu.sync_copy(pe_hbm, pe_v)
        w_base = wid * seqs_per_worker * SEQ_LEN

        def chunk_body(c, carry):
            base = w_base + c * rows_per_chunk
            pltpu.sync_copy(ids_hbm.at[pl.ds(base, rows_per_chunk)], idx_v)
            pltpu.async_copy(table_hbm.at[idx_v], rows_v, sem).wait()

            def pos_body(s, carry2):
                for q in range(CHUNK_SEQS):
                    r = q * SEQ_LEN + s
                    for d in range(D_MODEL // LANES):
                        sl = pl.ds(d * LANES, LANES)
                        rows_v[r, sl] = rows_v[r, sl] * 9.0 + pe_v[s, sl]
                return carry2

            lax.fori_loop(0, SEQ_LEN, pos_body, 0, unroll=False)
            pltpu.sync_copy(rows_v, out_hbm.at[pl.ds(base, rows_per_chunk)])
            return carry

        lax.fori_loop(0, n_chunks, chunk_body, 0, unroll=False)

    return body


@jax.jit
def kernel(input_ids, table):
    batch, seq = input_ids.shape
    ids_flat = input_ids.reshape(batch * seq)
    out = _build(batch)(ids_flat, table, _PE)
    return out.reshape(batch, seq, D_MODEL)
